# Initial kernel scaffold; baseline (speedup 1.0000x reference)
import functools

import jax
import jax.numpy as jnp
from jax import lax
from jax.experimental import pallas as pl
from jax.experimental.pallas import tpu as pltpu
from jax.experimental.pallas import tpu_sc as plsc

NC_OUT = 8192
NF_IN = 16384
FDIM = 256
FH = FDIM // 2
NTILES = 16
BATCH = 128


def _sc_spmm(nbatches):
    c_per_tile = nbatches * BATCH

    def body(x2, rows_h, cols_h, vals_h, out_h, ibuf, vbuf, rbuf, gbuf, acc, sem):
        core = lax.axis_index("c")
        tid = lax.axis_index("s")

        zero16 = jnp.zeros((16,), jnp.float32)

        def zg(i, carry):
            j = i // 8
            f = lax.rem(i, 8)
            gbuf[j, pl.ds(f * 16, 16)] = zero16
            return carry

        lax.fori_loop(0, BATCH * 8, zg, 0)
        for r in range(4):
            pltpu.sync_copy(gbuf, acc.at[pl.ds(tid * 512 + r * 128, 128)])
        plsc.subcore_barrier()

        def step(gi, carry):
            base = tid * c_per_tile + gi * BATCH
            pltpu.sync_copy(cols_h.at[pl.ds(base, BATCH)], ibuf)
            pltpu.sync_copy(vals_h.at[pl.ds(base, BATCH)], vbuf)
            pltpu.sync_copy(rows_h.at[pl.ds(base, BATCH)], rbuf)
            for g in range(BATCH // 16):
                c16 = ibuf[pl.ds(g * 16, 16)]
                ibuf[pl.ds(g * 16, 16)] = c16 * 2 + core
            pltpu.async_copy(x2.at[ibuf], gbuf, sem).wait()

            def scale(j, carry2):
                vj = plsc.load_gather(vbuf, [jnp.full((16,), j, jnp.int32)])
                for f in range(FH // 16):
                    g16 = gbuf[j, pl.ds(f * 16, 16)]
                    gbuf[j, pl.ds(f * 16, 16)] = g16 * vj
                return carry2

            lax.fori_loop(0, BATCH, scale, 0)

            pltpu.sync_copy(gbuf, acc.at[rbuf], add=True)
            return carry

        lax.fori_loop(0, nbatches, step, 0)
        plsc.subcore_barrier()

        for r in range(4):
            row0 = tid * 512 + r * 128
            pltpu.sync_copy(
                acc.at[pl.ds(row0, 128)],
                out_h.at[pl.ds(row0, 128), pl.ds(core * FH, FH)],
            )

    return body


@jax.jit
def kernel(x_fine, rows, cols, vals):
    nnz = rows.shape[0]
    c_per_tile = -(-nnz // (NTILES * BATCH)) * BATCH
    nnz_pad = c_per_tile * NTILES
    pad = nnz_pad - nnz
    rows_p = jnp.pad(rows, (0, pad))
    cols_p = jnp.pad(cols, (0, pad))
    vals_p = jnp.pad(vals, (0, pad))
    x2 = x_fine.reshape(2 * NF_IN, FH)

    mesh = plsc.VectorSubcoreMesh(core_axis_name="c", subcore_axis_name="s")
    f = pl.kernel(
        _sc_spmm(c_per_tile // BATCH),
        mesh=mesh,
        out_type=jax.ShapeDtypeStruct((NC_OUT, FDIM), jnp.float32),
        scratch_types=[
            pltpu.VMEM((BATCH,), jnp.int32),
            pltpu.VMEM((BATCH,), jnp.float32),
            pltpu.VMEM((BATCH,), jnp.int32),
            pltpu.VMEM((BATCH, FH), jnp.float32),
            pltpu.VMEM_SHARED((NC_OUT, FH), jnp.float32),
            pltpu.SemaphoreType.DMA,
        ],
    )
    return f(x2, rows_p, cols_p, vals_p)


# speedup vs baseline: 5.4650x; 5.4650x over previous
import functools

import jax
import jax.numpy as jnp
from jax import lax
from jax.experimental import pallas as pl
from jax.experimental.pallas import tpu as pltpu
from jax.experimental.pallas import tpu_sc as plsc

NC_OUT = 8192
NF_IN = 16384
FDIM = 256
FH = FDIM // 2
NTILES = 16
BATCH = 128


def _sc_spmm(nbatches):
    c_per_tile = nbatches * BATCH

    def body(x2, rows_h, cols_h, vals_h, out_h, ibuf, vbuf, rbuf, gbuf, acc, sem):
        core = lax.axis_index("c")
        tid = lax.axis_index("s")

        zero16 = jnp.zeros((16,), jnp.float32)

        def zg(i, carry):
            j = i // 8
            f = lax.rem(i, 8)
            gbuf[j, pl.ds(f * 16, 16)] = zero16
            return carry

        lax.fori_loop(0, BATCH * 8, zg, 0)
        for r in range(4):
            pltpu.sync_copy(gbuf, acc.at[pl.ds(tid * 512 + r * 128, 128)])
        plsc.subcore_barrier()

        def step(gi, carry):
            base = tid * c_per_tile + gi * BATCH
            pltpu.sync_copy(cols_h.at[pl.ds(base, BATCH)], ibuf)
            pltpu.sync_copy(vals_h.at[pl.ds(base, BATCH)], vbuf)
            pltpu.sync_copy(rows_h.at[pl.ds(base, BATCH)], rbuf)
            for g in range(BATCH // 16):
                c16 = ibuf[pl.ds(g * 16, 16)]
                ibuf[pl.ds(g * 16, 16)] = c16 * 2 + core
            pltpu.async_copy(x2.at[ibuf], gbuf, sem).wait()

            def scale(g, carry2):
                vgrp = vbuf[pl.ds(g * 16, 16)]
                for jm in range(16):
                    j = g * 16 + jm
                    vj = vgrp[jm]
                    for f in range(FH // 16):
                        g16 = gbuf[j, pl.ds(f * 16, 16)]
                        gbuf[j, pl.ds(f * 16, 16)] = g16 * vj
                return carry2

            lax.fori_loop(0, BATCH // 16, scale, 0)

            pltpu.sync_copy(gbuf, acc.at[rbuf], add=True)
            return carry

        lax.fori_loop(0, nbatches, step, 0)
        plsc.subcore_barrier()

        for r in range(4):
            row0 = tid * 512 + r * 128
            pltpu.sync_copy(
                acc.at[pl.ds(row0, 128)],
                out_h.at[pl.ds(row0, 128), pl.ds(core * FH, FH)],
            )

    return body


@jax.jit
def kernel(x_fine, rows, cols, vals):
    nnz = rows.shape[0]
    c_per_tile = -(-nnz // (NTILES * BATCH)) * BATCH
    nnz_pad = c_per_tile * NTILES
    pad = nnz_pad - nnz
    rows_p = jnp.pad(rows, (0, pad))
    cols_p = jnp.pad(cols, (0, pad))
    vals_p = jnp.pad(vals, (0, pad))
    x2 = x_fine.reshape(2 * NF_IN, FH)

    mesh = plsc.VectorSubcoreMesh(core_axis_name="c", subcore_axis_name="s")
    f = pl.kernel(
        _sc_spmm(c_per_tile // BATCH),
        mesh=mesh,
        out_type=jax.ShapeDtypeStruct((NC_OUT, FDIM), jnp.float32),
        scratch_types=[
            pltpu.VMEM((BATCH,), jnp.int32),
            pltpu.VMEM((BATCH,), jnp.float32),
            pltpu.VMEM((BATCH,), jnp.int32),
            pltpu.VMEM((BATCH, FH), jnp.float32),
            pltpu.VMEM_SHARED((NC_OUT, FH), jnp.float32),
            pltpu.SemaphoreType.DMA,
        ],
    )
    return f(x2, rows_p, cols_p, vals_p)
